# baseline (speedup 1.0000x reference)
import functools

import jax
import jax.numpy as jnp
from jax import lax
from jax.experimental import pallas as pl
from jax.experimental.pallas import tpu as pltpu
from jax.experimental.pallas import tpu_sc as plsc

N_NODES = 10000
N_EDGES = 320000
D = 128

NC = 2
NS = 16
L = 16

CHUNK = 96
NCHUNK0 = 168
NCHUNK1 = 48
TOTAL_CHUNKS = NS * (NCHUNK0 + NCHUNK1)
CHUNK_PAD = TOTAL_CHUNKS + (NCHUNK0 - NCHUNK1)
N_PAD = 10240
ROWS_PER_TILE = N_PAD // NS
ZROWS = 128


def _matmul(x, W):
    def mm_kernel(x_ref, w_ref, o_ref):
        o_ref[...] = jnp.dot(x_ref[...], w_ref[...],
                             preferred_element_type=jnp.float32)

    return pl.pallas_call(
        mm_kernel,
        grid=(10,),
        in_specs=[
            pl.BlockSpec((1000, D), lambda i: (i, 0)),
            pl.BlockSpec((D, D), lambda i: (0, 0)),
        ],
        out_specs=pl.BlockSpec((1000, D), lambda i: (i, 0)),
        out_shape=jax.ShapeDtypeStruct((N_NODES, D), jnp.float32),
    )(x, W)


_SC_MESH = plsc.VectorSubcoreMesh(
    core_axis_name="c", subcore_axis_name="s", num_cores=NC, num_subcores=NS)


@functools.partial(
    pl.kernel,
    mesh=_SC_MESH,
    out_type=jax.ShapeDtypeStruct((NC, N_PAD, D), jnp.float32),
    scratch_types=[
        pltpu.VMEM((NCHUNK0, CHUNK), jnp.int32),
        pltpu.VMEM((CHUNK,), jnp.int32),
        pltpu.VMEM((CHUNK,), jnp.int32),
        pltpu.VMEM((CHUNK,), jnp.float32),
        pltpu.VMEM((CHUNK,), jnp.float32),
        pltpu.VMEM((CHUNK, D), jnp.float32),
        pltpu.VMEM((CHUNK, D), jnp.float32),
        pltpu.VMEM_SHARED((N_PAD, D), jnp.float32),
        pltpu.SemaphoreType.DMA,
        pltpu.SemaphoreType.DMA,
        pltpu.SemaphoreType.DMA,
        pltpu.SemaphoreType.DMA,
        pltpu.SemaphoreType.DMA,
        pltpu.SemaphoreType.DMA,
        pltpu.SemaphoreType.DMA,
        pltpu.SemaphoreType.DMA,
    ],
)
def _sc_aggregate(support_hbm, src_hbm, dst_hbm, w_hbm, out_hbm,
                  src_v, dst0, dst1, w0, w1, rows0, rows1, accum,
                  dsem0, dsem1, wsem0, wsem1, gsem0, gsem1, ssem0, ssem1):
    c = lax.axis_index("c")
    s = lax.axis_index("s")
    base_chunk = jnp.where(c == 0, s * NCHUNK0, NS * NCHUNK0 + s * NCHUNK1)
    npair = jnp.where(c == 0, NCHUNK0 // 2, NCHUNK1 // 2)

    pltpu.sync_copy(src_hbm.at[pl.ds(base_chunk, NCHUNK0)], src_v)

    def zero_row(i, _):
        for cc in range(D // L):
            rows0[i, pl.ds(cc * L, L)] = jnp.zeros((L,), jnp.float32)
        return 0
    lax.fori_loop(0, ZROWS, zero_row, 0)
    row0 = s * ROWS_PER_TILE
    for b in range(ROWS_PER_TILE // ZROWS):
        pltpu.sync_copy(rows0.at[pl.ds(0, ZROWS)],
                        accum.at[pl.ds(row0 + b * ZROWS, ZROWS)])

    pltpu.async_copy(dst_hbm.at[base_chunk], dst0, dsem0)
    pltpu.async_copy(dst_hbm.at[base_chunk + 1], dst1, dsem1)
    pltpu.async_copy(w_hbm.at[base_chunk], w0, wsem0)
    pltpu.async_copy(w_hbm.at[base_chunk + 1], w1, wsem1)
    pltpu.async_copy(support_hbm.at[src_v.at[0]], rows0, gsem0)
    pltpu.async_copy(support_hbm.at[src_v.at[1]], rows1, gsem1)
    plsc.subcore_barrier()

    dummy_rows = support_hbm.at[pl.ds(0, CHUNK)]
    dummy_dst = dst_hbm.at[0]
    dummy_w = w_hbm.at[0]

    def scale(rows, w_ref):
        def scale_group(g, _):
            wv = w_ref[pl.ds(g * L, L)]
            for j in range(L):
                wvec = jnp.full((L,), wv[j], jnp.float32)
                r = g * L + j
                for cc in range(D // L):
                    sl = pl.ds(cc * L, L)
                    rows[r, sl] = rows[r, sl] * wvec
            return 0
        lax.fori_loop(0, CHUNK // L, scale_group, 0)

    def pair(g, _):
        e0 = 2 * g
        e1 = e0 + 1
        pltpu.make_async_copy(dummy_rows, rows0, gsem0).wait()
        pltpu.make_async_copy(dummy_dst, dst0, dsem0).wait()
        pltpu.make_async_copy(dummy_w, w0, wsem0).wait()
        scale(rows0, w0)
        sd0 = pltpu.async_copy(rows0, accum.at[dst0], ssem0, add=True)
        pltpu.make_async_copy(dummy_rows, rows1, gsem1).wait()
        pltpu.make_async_copy(dummy_dst, dst1, dsem1).wait()
        pltpu.make_async_copy(dummy_w, w1, wsem1).wait()
        scale(rows1, w1)
        sd1 = pltpu.async_copy(rows1, accum.at[dst1], ssem1, add=True)
        sd0.wait()

        @pl.when(g < npair - 1)
        def _():
            pltpu.async_copy(dst_hbm.at[base_chunk + e0 + 2], dst0, dsem0)
            pltpu.async_copy(w_hbm.at[base_chunk + e0 + 2], w0, wsem0)
            pltpu.async_copy(support_hbm.at[src_v.at[e0 + 2]], rows0, gsem0)
        sd1.wait()

        @pl.when(g < npair - 1)
        def _():
            pltpu.async_copy(dst_hbm.at[base_chunk + e1 + 2], dst1, dsem1)
            pltpu.async_copy(w_hbm.at[base_chunk + e1 + 2], w1, wsem1)
            pltpu.async_copy(support_hbm.at[src_v.at[e1 + 2]], rows1, gsem1)
        return 0
    lax.fori_loop(0, npair, pair, 0)
    plsc.subcore_barrier()

    pltpu.sync_copy(accum.at[pl.ds(row0, ROWS_PER_TILE)],
                    out_hbm.at[c, pl.ds(row0, ROWS_PER_TILE)])


def _add_relu(partials):
    def ar_kernel(p_ref, o_ref):
        o_ref[...] = jnp.maximum(p_ref[0] + p_ref[1], 0.0)

    return pl.pallas_call(
        ar_kernel,
        grid=(10,),
        in_specs=[pl.BlockSpec((NC, 1000, D), lambda i: (0, i, 0))],
        out_specs=pl.BlockSpec((1000, D), lambda i: (i, 0)),
        out_shape=jax.ShapeDtypeStruct((N_NODES, D), jnp.float32),
    )(partials)


def kernel(x, edge_index, edge_weight, W):
    support = _matmul(x, W)
    dst = edge_index[0].astype(jnp.int32)
    src = edge_index[1].astype(jnp.int32)
    pad = CHUNK_PAD * CHUNK - N_EDGES
    src = jnp.pad(src, (0, pad)).reshape(CHUNK_PAD, CHUNK)
    dst = jnp.pad(dst, (0, pad)).reshape(CHUNK_PAD, CHUNK)
    w = jnp.pad(edge_weight, (0, pad)).reshape(CHUNK_PAD, CHUNK)
    partials = _sc_aggregate(support, src, dst, w)
    return _add_relu(partials[:, :N_NODES])


# speedup vs baseline: 1.1747x; 1.1747x over previous
import functools

import jax
import jax.numpy as jnp
from jax import lax
from jax.experimental import pallas as pl
from jax.experimental.pallas import tpu as pltpu
from jax.experimental.pallas import tpu_sc as plsc

N_NODES = 10000
N_EDGES = 320000
D = 128

NC = 2
NS = 16
L = 16
NW = NC * NS

CHUNK = 64
NCHUNK = 160
NBUF = 4
NQUAD = NCHUNK // NBUF
E_PAD = NW * NCHUNK * CHUNK
TOTAL_CHUNKS = NW * NCHUNK
N_PAD = 10240
ROWS_PER_TILE = N_PAD // NS
ZROWS = 128


def _matmul(x, W):
    def mm_kernel(x_ref, w_ref, o_ref):
        o_ref[...] = jnp.dot(x_ref[...], w_ref[...],
                             preferred_element_type=jnp.float32)

    return pl.pallas_call(
        mm_kernel,
        grid=(10,),
        in_specs=[
            pl.BlockSpec((1000, D), lambda i: (i, 0)),
            pl.BlockSpec((D, D), lambda i: (0, 0)),
        ],
        out_specs=pl.BlockSpec((1000, D), lambda i: (i, 0)),
        out_shape=jax.ShapeDtypeStruct((N_NODES, D), jnp.float32),
    )(x, W)


_SC_MESH = plsc.VectorSubcoreMesh(
    core_axis_name="c", subcore_axis_name="s", num_cores=NC, num_subcores=NS)


@functools.partial(
    pl.kernel,
    mesh=_SC_MESH,
    out_type=jax.ShapeDtypeStruct((NC, N_PAD, D), jnp.float32),
    scratch_types=(
        [pltpu.VMEM((NCHUNK // 2, 2 * CHUNK), jnp.int32)]
        + [pltpu.VMEM((NBUF, CHUNK), jnp.int32)]
        + [pltpu.VMEM((NBUF, CHUNK), jnp.float32)]
        + [pltpu.VMEM((NBUF, CHUNK, D), jnp.float32)]
        + [pltpu.VMEM_SHARED((N_PAD, D), jnp.float32)]
        + [pltpu.SemaphoreType.DMA for _ in range(4 * NBUF)]
    ),
)
def _sc_aggregate(support_hbm, src_hbm, dst_hbm, w_hbm, out_hbm,
                  src_v, *rest):
    dst2 = rest[0]
    w2 = rest[1]
    rows3 = rest[2]
    rowsb = [rows3.at[b] for b in range(NBUF)]
    accum = rest[3]
    sems = rest[4:]
    dsem = sems[0:NBUF]
    wsem = sems[NBUF:2 * NBUF]
    gsem = sems[2 * NBUF:3 * NBUF]
    ssem = sems[3 * NBUF:4 * NBUF]

    c = lax.axis_index("c")
    s = lax.axis_index("s")
    wid = c * NS + s
    base_chunk = wid * NCHUNK

    pltpu.sync_copy(src_hbm.at[wid], src_v)

    def zero_row(i, _):
        for cc in range(D // L):
            rows3[0, i, pl.ds(cc * L, L)] = jnp.zeros((L,), jnp.float32)
            rows3[1, i, pl.ds(cc * L, L)] = jnp.zeros((L,), jnp.float32)
        return 0
    lax.fori_loop(0, CHUNK, zero_row, 0)
    row0 = s * ROWS_PER_TILE
    for b in range(ROWS_PER_TILE // CHUNK):
        pltpu.sync_copy(rowsb[b % 2], accum.at[pl.ds(row0 + b * CHUNK, CHUNK)])

    for b in range(NBUF):
        pltpu.async_copy(dst_hbm.at[base_chunk + b], dst2.at[b], dsem[b])
        pltpu.async_copy(w_hbm.at[base_chunk + b], w2.at[b], wsem[b])
        pltpu.async_copy(
            support_hbm.at[src_v.at[b // 2, pl.ds((b % 2) * CHUNK, CHUNK)]],
            rowsb[b], gsem[b])
    plsc.subcore_barrier()

    dummy_rows = support_hbm.at[pl.ds(0, CHUNK)]
    dummy_dst = dst_hbm.at[0]
    dummy_w = w_hbm.at[0]
    dst_rows = [dst2.at[b] for b in range(NBUF)]
    w_rows = [w2.at[b] for b in range(NBUF)]

    def scale(b, wrow):
        def scale_group(g, _):
            wv = w2[wrow, pl.ds(g * L, L)]
            for j in range(L):
                wvec = jnp.full((L,), wv[j], jnp.float32)
                r = g * L + j
                for cc in range(D // L):
                    sl = pl.ds(cc * L, L)
                    rows3[b, r, sl] = rows3[b, r, sl] * wvec
            return 0
        lax.fori_loop(0, CHUNK // L, scale_group, 0)

    def quad(q, _):
        e_base = NBUF * q
        sds = []
        for b in range(NBUF):
            pltpu.make_async_copy(dummy_rows, rowsb[b], gsem[b]).wait()
            pltpu.make_async_copy(dummy_dst, dst_rows[b], dsem[b]).wait()
            pltpu.make_async_copy(dummy_w, w_rows[b], wsem[b]).wait()
            scale(b, b)
            sds.append(pltpu.async_copy(
                rowsb[b], accum.at[dst_rows[b]], ssem[b], add=True))
        for b in range(NBUF):
            sds[b].wait()
            e_next = e_base + b + NBUF

            @pl.when(q < NQUAD - 1)
            def _(b=b, e_next=e_next):
                pltpu.async_copy(
                    dst_hbm.at[base_chunk + e_next], dst_rows[b], dsem[b])
                pltpu.async_copy(
                    w_hbm.at[base_chunk + e_next], w_rows[b], wsem[b])
                pltpu.async_copy(
                    support_hbm.at[
                        src_v.at[e_next // 2, pl.ds((e_next % 2) * CHUNK,
                                                    CHUNK)]],
                    rowsb[b], gsem[b])
        return 0
    lax.fori_loop(0, NQUAD, quad, 0)
    plsc.subcore_barrier()

    pltpu.sync_copy(accum.at[pl.ds(row0, ROWS_PER_TILE)],
                    out_hbm.at[c, pl.ds(row0, ROWS_PER_TILE)])


def _add_relu(partials):
    def ar_kernel(p_ref, o_ref):
        o_ref[...] = jnp.maximum(p_ref[0] + p_ref[1], 0.0)

    return pl.pallas_call(
        ar_kernel,
        grid=(10,),
        in_specs=[pl.BlockSpec((NC, 1000, D), lambda i: (0, i, 0))],
        out_specs=pl.BlockSpec((1000, D), lambda i: (i, 0)),
        out_shape=jax.ShapeDtypeStruct((N_NODES, D), jnp.float32),
    )(partials)


def kernel(x, edge_index, edge_weight, W):
    support = _matmul(x, W)
    dst = edge_index[0].astype(jnp.int32)
    src = edge_index[1].astype(jnp.int32)
    pad = E_PAD - N_EDGES
    src = jnp.pad(src, (0, pad)).reshape(NW, NCHUNK // 2, 2 * CHUNK)
    dst = jnp.pad(dst, (0, pad)).reshape(TOTAL_CHUNKS, CHUNK)
    w = jnp.pad(edge_weight, (0, pad)).reshape(TOTAL_CHUNKS, CHUNK)
    partials = _sc_aggregate(support, src, dst, w)
    return _add_relu(partials[:, :N_NODES])


# speedup vs baseline: 1.4001x; 1.1918x over previous
import functools

import jax
import jax.numpy as jnp
from jax import lax
from jax.experimental import pallas as pl
from jax.experimental.pallas import tpu as pltpu
from jax.experimental.pallas import tpu_sc as plsc

N_NODES = 10000
N_EDGES = 320000
D = 128

NC = 2
NS = 16
L = 16
NW = NC * NS

EDGES_PER_TILE = N_EDGES // NW
CHUNK = 80
NCHUNK = EDGES_PER_TILE // CHUNK
N_PAD = 10240
ROWS_PER_TILE = N_PAD // NS
ZROWS = 128


def _matmul(x, W):
    def mm_kernel(x_ref, w_ref, o_ref):
        o_ref[...] = jnp.dot(x_ref[...], w_ref[...],
                             preferred_element_type=jnp.float32)

    return pl.pallas_call(
        mm_kernel,
        grid=(10,),
        in_specs=[
            pl.BlockSpec((1000, D), lambda i: (i, 0)),
            pl.BlockSpec((D, D), lambda i: (0, 0)),
        ],
        out_specs=pl.BlockSpec((1000, D), lambda i: (i, 0)),
        out_shape=jax.ShapeDtypeStruct((N_NODES, D), jnp.float32),
    )(x, W)


_SC_MESH = plsc.VectorSubcoreMesh(
    core_axis_name="c", subcore_axis_name="s", num_cores=NC, num_subcores=NS)


@functools.partial(
    pl.kernel,
    mesh=_SC_MESH,
    out_type=jax.ShapeDtypeStruct((NC, N_PAD, D), jnp.float32),
    scratch_types=[
        pltpu.VMEM((CHUNK,), jnp.int32),
        pltpu.VMEM((CHUNK,), jnp.int32),
        pltpu.VMEM((CHUNK,), jnp.float32),
        pltpu.VMEM((CHUNK, D), jnp.float32),
        pltpu.VMEM((ZROWS, D), jnp.float32),
        pltpu.VMEM_SHARED((N_PAD, D), jnp.float32),
        pltpu.SemaphoreType.DMA,
    ],
)
def _sc_aggregate(support_hbm, src_hbm, dst_hbm, w_hbm, out_hbm,
                  src_v, dst_v, w_v, rows_v, z_v, accum, sem):
    c = lax.axis_index("c")
    s = lax.axis_index("s")

    def zero_row(i, _):
        for cc in range(D // L):
            z_v[i, pl.ds(cc * L, L)] = jnp.zeros((L,), jnp.float32)
        return 0
    lax.fori_loop(0, ZROWS, zero_row, 0)
    row0 = s * ROWS_PER_TILE
    for b in range(ROWS_PER_TILE // ZROWS):
        pltpu.sync_copy(z_v, accum.at[pl.ds(row0 + b * ZROWS, ZROWS)])
    plsc.subcore_barrier()

    base_e = (c * NS + s) * EDGES_PER_TILE

    def edge_chunk(i, _):
        off = base_e + i * CHUNK
        pltpu.sync_copy(src_hbm.at[pl.ds(off, CHUNK)], src_v)
        pltpu.sync_copy(dst_hbm.at[pl.ds(off, CHUNK)], dst_v)
        pltpu.sync_copy(w_hbm.at[pl.ds(off, CHUNK)], w_v)
        pltpu.async_copy(support_hbm.at[src_v], rows_v, sem).wait()

        def scale_group(g, _):
            wv = w_v[pl.ds(g * L, L)]
            for j in range(L):
                wvec = jnp.full((L,), wv[j], jnp.float32)
                r = g * L + j
                for cc in range(D // L):
                    sl = pl.ds(cc * L, L)
                    rows_v[r, sl] = rows_v[r, sl] * wvec
            return 0
        lax.fori_loop(0, CHUNK // L, scale_group, 0)

        pltpu.sync_copy(rows_v, accum.at[dst_v], add=True)
        return 0
    lax.fori_loop(0, NCHUNK, edge_chunk, 0)
    plsc.subcore_barrier()

    pltpu.sync_copy(accum.at[pl.ds(row0, ROWS_PER_TILE)],
                    out_hbm.at[c, pl.ds(row0, ROWS_PER_TILE)])


def _add_relu(partials):
    def ar_kernel(p_ref, o_ref):
        o_ref[...] = jnp.maximum(p_ref[0] + p_ref[1], 0.0)

    return pl.pallas_call(
        ar_kernel,
        grid=(10,),
        in_specs=[pl.BlockSpec((NC, 1000, D), lambda i: (0, i, 0))],
        out_specs=pl.BlockSpec((1000, D), lambda i: (i, 0)),
        out_shape=jax.ShapeDtypeStruct((N_NODES, D), jnp.float32),
    )(partials)


def kernel(x, edge_index, edge_weight, W):
    support = _matmul(x, W)
    dst = edge_index[0].astype(jnp.int32)
    src = edge_index[1].astype(jnp.int32)
    partials = _sc_aggregate(support, src, dst, edge_weight)
    return _add_relu(partials[:, :N_NODES])


# speedup vs baseline: 2.0106x; 1.4361x over previous
import functools

import jax
import jax.numpy as jnp
from jax import lax
from jax.experimental import pallas as pl
from jax.experimental.pallas import tpu as pltpu
from jax.experimental.pallas import tpu_sc as plsc

N_NODES = 10000
N_EDGES = 320000
D = 128

NC = 2
NS = 16
L = 16

CHUNK = 64
NCH0 = 236
NCH1 = 77
NBUF = 4
NQUAD = NCH0 // NBUF
E0 = NS * NCH0 * CHUNK
E1 = NS * NCH1 * CHUNK
N_PAD = 10112
ROWS_PER_TILE = N_PAD // NS


def _matmul(x, W):
    def mm_kernel(x_ref, w_ref, o_ref):
        o_ref[...] = jnp.dot(x_ref[...], w_ref[...],
                             preferred_element_type=jnp.float32)

    return pl.pallas_call(
        mm_kernel,
        grid=(10,),
        in_specs=[
            pl.BlockSpec((1000, D), lambda i: (i, 0)),
            pl.BlockSpec((D, D), lambda i: (0, 0)),
        ],
        out_specs=pl.BlockSpec((1000, D), lambda i: (i, 0)),
        out_shape=jax.ShapeDtypeStruct((N_NODES, D), jnp.float32),
    )(x, W)


_SC_MESH = plsc.VectorSubcoreMesh(
    core_axis_name="c", subcore_axis_name="s", num_cores=NC, num_subcores=NS)


@functools.partial(
    pl.kernel,
    mesh=_SC_MESH,
    out_type=jax.ShapeDtypeStruct((NC, N_PAD, D), jnp.float32),
    scratch_types=(
        [pltpu.VMEM((NCH0 // 2, 2 * CHUNK), jnp.int32)]
        + [pltpu.VMEM((NBUF, CHUNK), jnp.int32)]
        + [pltpu.VMEM((NBUF, CHUNK), jnp.float32)]
        + [pltpu.VMEM((NBUF, CHUNK, D), jnp.float32)]
        + [pltpu.VMEM_SHARED((N_PAD, D), jnp.float32)]
        + [pltpu.SemaphoreType.DMA for _ in range(4 * NBUF)]
    ),
)
def _sc_aggregate(support_hbm, src0_hbm, dst0_hbm, w0_hbm,
                  src1_hbm, dst1_hbm, w1_hbm, out_hbm,
                  src_v, dst2, w2, rows3, accum, *sems):
    dsem = sems[0:NBUF]
    wsem = sems[NBUF:2 * NBUF]
    gsem = sems[2 * NBUF:3 * NBUF]
    ssem = sems[3 * NBUF:4 * NBUF]

    c = lax.axis_index("c")
    s = lax.axis_index("s")

    def zero_row(i, _):
        for cc in range(D // L):
            rows3[0, i, pl.ds(cc * L, L)] = jnp.zeros((L,), jnp.float32)
            rows3[1, i, pl.ds(cc * L, L)] = jnp.zeros((L,), jnp.float32)
        return 0
    lax.fori_loop(0, CHUNK, zero_row, 0)
    row0 = s * ROWS_PER_TILE
    for b in range(ROWS_PER_TILE // CHUNK):
        pltpu.sync_copy(rows3.at[b % 2],
                        accum.at[pl.ds(row0 + b * CHUNK, CHUNK)])
    rem = ROWS_PER_TILE % CHUNK
    if rem:
        pltpu.sync_copy(
            rows3.at[0, pl.ds(0, rem)],
            accum.at[pl.ds(row0 + (ROWS_PER_TILE // CHUNK) * CHUNK, rem)])

    base0 = s * NCH0

    @pl.when(c == 0)
    def _():
        pltpu.sync_copy(src0_hbm.at[s], src_v)
        for b in range(NBUF):
            pltpu.async_copy(dst0_hbm.at[base0 + b], dst2.at[b], dsem[b])
            pltpu.async_copy(w0_hbm.at[base0 + b], w2.at[b], wsem[b])
            pltpu.async_copy(
                support_hbm.at[src_v.at[b // 2, pl.ds((b % 2) * CHUNK, CHUNK)]],
                rows3.at[b], gsem[b])
    plsc.subcore_barrier()

    dummy_rows = support_hbm.at[pl.ds(0, CHUNK)]
    dummy_dst = dst0_hbm.at[0]
    dummy_w = w0_hbm.at[0]

    def scale(brow, wrow):
        def scale_group(g, _):
            wv = w2[wrow, pl.ds(g * L, L)]
            for j in range(L):
                wvec = jnp.full((L,), wv[j], jnp.float32)
                r = g * L + j
                for cc in range(D // L):
                    sl = pl.ds(cc * L, L)
                    rows3[brow, r, sl] = rows3[brow, r, sl] * wvec
            return 0
        lax.fori_loop(0, CHUNK // L, scale_group, 0)

    @pl.when(c == 0)
    def _():
        def quad(q, _):
            e_base = NBUF * q
            sds = []
            for b in range(NBUF):
                pltpu.make_async_copy(dummy_rows, rows3.at[b], gsem[b]).wait()
                pltpu.make_async_copy(dummy_dst, dst2.at[b], dsem[b]).wait()
                pltpu.make_async_copy(dummy_w, w2.at[b], wsem[b]).wait()
                scale(b, b)
                sds.append(pltpu.async_copy(
                    rows3.at[b], accum.at[dst2.at[b]], ssem[b], add=True))
            for b in range(NBUF):
                sds[b].wait()
                e_next = e_base + b + NBUF

                @pl.when(q < NQUAD - 1)
                def _(b=b, e_next=e_next):
                    pltpu.async_copy(
                        dst0_hbm.at[base0 + e_next], dst2.at[b], dsem[b])
                    pltpu.async_copy(
                        w0_hbm.at[base0 + e_next], w2.at[b], wsem[b])
                    pltpu.async_copy(
                        support_hbm.at[
                            src_v.at[e_next // 2,
                                     pl.ds((e_next % 2) * CHUNK, CHUNK)]],
                        rows3.at[b], gsem[b])
            return 0
        lax.fori_loop(0, NQUAD, quad, 0)

    @pl.when(c == 1)
    def _():
        def edge_chunk(i, _):
            off = (s * NCH1 + i) * CHUNK
            pltpu.sync_copy(src1_hbm.at[pl.ds(off, CHUNK)], dst2.at[1])
            pltpu.sync_copy(dst1_hbm.at[pl.ds(off, CHUNK)], dst2.at[0])
            pltpu.sync_copy(w1_hbm.at[pl.ds(off, CHUNK)], w2.at[0])
            pltpu.async_copy(support_hbm.at[dst2.at[1]], rows3.at[0],
                             gsem[0]).wait()
            scale(0, 0)
            pltpu.sync_copy(rows3.at[0], accum.at[dst2.at[0]], add=True)
            return 0
        lax.fori_loop(0, NCH1, edge_chunk, 0)
    plsc.subcore_barrier()

    pltpu.sync_copy(accum.at[pl.ds(row0, ROWS_PER_TILE)],
                    out_hbm.at[c, pl.ds(row0, ROWS_PER_TILE)])


def _add_relu(partials):
    def ar_kernel(p_ref, o_ref):
        o_ref[...] = jnp.maximum(p_ref[0] + p_ref[1], 0.0)

    return pl.pallas_call(
        ar_kernel,
        grid=(10,),
        in_specs=[pl.BlockSpec((NC, 1000, D), lambda i: (0, i, 0))],
        out_specs=pl.BlockSpec((1000, D), lambda i: (i, 0)),
        out_shape=jax.ShapeDtypeStruct((N_NODES, D), jnp.float32),
    )(partials)


def kernel(x, edge_index, edge_weight, W):
    support = _matmul(x, W)
    dst = edge_index[0].astype(jnp.int32)
    src = edge_index[1].astype(jnp.int32)
    w = edge_weight
    src0 = src[:E0].reshape(NS, NCH0 // 2, 2 * CHUNK)
    dst0 = dst[:E0].reshape(NS * NCH0, CHUNK)
    w0 = w[:E0].reshape(NS * NCH0, CHUNK)
    pad1 = E0 + E1 - N_EDGES
    src1 = jnp.pad(src[E0:], (0, pad1))
    dst1 = jnp.pad(dst[E0:], (0, pad1))
    w1 = jnp.pad(w[E0:], (0, pad1))
    partials = _sc_aggregate(support, src0, dst0, w0, src1, dst1, w1)
    return _add_relu(partials[:, :N_NODES])
